# baseline (speedup 1.0000x reference)
import jax
import jax.numpy as jnp
from jax import lax
from jax.experimental import pallas as pl
from jax.experimental.pallas import tpu as pltpu
from jax.experimental.pallas import tpu_sc as plsc

N = 10000
E = 320000
D = 128
G = 32
NC = 2
NS = 16
NW = NC * NS
K = 64
CH = 160
CQ = 16
NG = CH // CQ
TPE = K * CH
EP = NW * TPE
NPAD = 10240
RPT = NPAD // NS


def _deg_body(dstp_hbm, z1_hbm, degp_hbm, shared, dstb, ones):
    c = lax.axis_index("c")
    s = lax.axis_index("s")
    w = c * NS + s
    pltpu.sync_copy(z1_hbm, shared.at[pl.ds(s * RPT, RPT)])
    pltpu.sync_copy(dstp_hbm.at[w], dstb)
    for i in range(128 // 16):
        ones[pl.ds(i * 16, 16)] = jnp.ones((16,), jnp.float32)
    plsc.subcore_barrier()

    def chunk(j, carry):
        pltpu.sync_copy(ones, shared.at[dstb.at[j]], add=True)
        return carry

    lax.fori_loop(0, TPE // 128, chunk, 0)
    plsc.subcore_barrier()
    off = pl.multiple_of(c * NPAD + s * RPT, 8)
    pltpu.sync_copy(shared.at[pl.ds(s * RPT, RPT)],
                    degp_hbm.at[pl.ds(off, RPT)])


def _scat_body(hs_hbm, gixp_hbm, rlp_hbm, dstp_hbm, z2_hbm, out_hbm,
               shared, win, gixb, rlb, dstb, stage_a, stage_a2, stage_b,
               sem_g, sem_g2, sem_s):
    c = lax.axis_index("c")
    s = lax.axis_index("s")
    w = c * NS + s
    pltpu.sync_copy(z2_hbm, shared.at[pl.ds(s * RPT, RPT)])
    plsc.subcore_barrier()

    for q in range(NG):
        pltpu.sync_copy(gixp_hbm.at[w, pl.ds(q * CQ, CQ)], gixb)
        pltpu.sync_copy(rlp_hbm.at[w, pl.ds(q * CQ, CQ)], rlb)
        pltpu.sync_copy(dstp_hbm.at[w, pl.ds(q * CQ, CQ)], dstb)
        pltpu.async_copy(hs_hbm.at[gixb.at[0]], stage_a, sem_g)
        pltpu.async_copy(hs_hbm.at[gixb.at[1]], stage_a2, sem_g2)

        def halfbody(i, ga, sga, wslot):
            pltpu.make_async_copy(hs_hbm.at[gixb.at[i]], ga, sga).wait()
            pltpu.sync_copy(ga, win.at[pl.ds(wslot, K)])

            @pl.when(i < CQ - 2)
            def _():
                pltpu.async_copy(hs_hbm.at[gixb.at[i + 2]], ga, sga)

            @pl.when(i > 0)
            def _():
                pltpu.make_async_copy(
                    stage_b, shared.at[dstb.at[i - 1]], sem_s).wait()

            pltpu.sync_copy(win.at[rlb.at[i]], stage_b)
            pltpu.async_copy(stage_b, shared.at[dstb.at[i]], sem_s, add=True)

        def body(i, carry):
            halfbody(2 * i, stage_a, sem_g, s * 2 * K)
            halfbody(2 * i + 1, stage_a2, sem_g2, s * 2 * K + K)
            return carry

        lax.fori_loop(0, CQ // 2, body, 0)
        pltpu.make_async_copy(
            stage_b, shared.at[dstb.at[CQ - 1]], sem_s).wait()

    plsc.subcore_barrier()
    off = pl.multiple_of(c * NPAD + s * RPT, 8)
    pltpu.sync_copy(shared.at[pl.ds(s * RPT, RPT)],
                    out_hbm.at[pl.ds(off, RPT)])


_SC_CALLS = {}


def _sc_calls():
    if not _SC_CALLS:
        mesh = plsc.VectorSubcoreMesh(
            core_axis_name="c", subcore_axis_name="s",
            num_cores=NC, num_subcores=NS)
        _SC_CALLS["deg"] = pl.kernel(
            _deg_body,
            out_type=jax.ShapeDtypeStruct((NC * NPAD,), jnp.float32),
            mesh=mesh,
            scratch_types=[
                pltpu.VMEM_SHARED((NPAD,), jnp.float32),
                pltpu.VMEM((TPE // 128, 128), jnp.int32),
                pltpu.VMEM((128,), jnp.float32),
            ],
        )
        _SC_CALLS["scat"] = pl.kernel(
            _scat_body,
            out_type=jax.ShapeDtypeStruct((NC * NPAD, D), jnp.float32),
            mesh=mesh,
            scratch_types=[
                pltpu.VMEM_SHARED((NPAD, D), jnp.float32),
                pltpu.VMEM_SHARED((NS * 2 * K, D), jnp.float32),
                pltpu.VMEM((CQ, K), jnp.int32),
                pltpu.VMEM((CQ, K), jnp.int32),
                pltpu.VMEM((CQ, K), jnp.int32),
                pltpu.VMEM((K, D), jnp.float32),
                pltpu.VMEM((K, D), jnp.float32),
                pltpu.VMEM((K, D), jnp.float32),
                pltpu.SemaphoreType.DMA,
                pltpu.SemaphoreType.DMA,
                pltpu.SemaphoreType.DMA,
            ],
        )
    return _SC_CALLS


def _deg_call(dstp, zeros1):
    return _sc_calls()["deg"](dstp, zeros1)


def _scat_call(hs, gixp, rlp, dstp, zeros2):
    return _sc_calls()["scat"](hs, gixp, rlp, dstp, zeros2)


def _wprep_body(srcp_ref, gix_ref, rl_ref):
    sp = srcp_ref[...]
    lo = jnp.min(sp, axis=1, keepdims=True) & (-8)
    hi = jnp.max(sp, axis=1, keepdims=True)
    narrow = (hi - lo) < K
    it = lax.broadcasted_iota(jnp.int32, (NW * CH, K), 1)
    row = lax.broadcasted_iota(jnp.int32, (NW * CH, K), 0)
    slot = ((row // CH % NS) * 2 + row % CH % 2) * K
    gix_ref[...] = jnp.where(narrow, lo + it, sp)
    rl_ref[...] = jnp.where(narrow, sp - lo, it) + slot


_wprep_call = pl.pallas_call(
    _wprep_body,
    out_shape=[jax.ShapeDtypeStruct((NW * CH, K), jnp.int32)] * 2,
)


def _t1_body(x_ref, w_ref, dinv_ref, hw_ref, hs_ref):
    hw = jnp.dot(x_ref[...], w_ref[...], preferred_element_type=jnp.float32)
    hw_ref[...] = hw
    hs_ref[:N, :] = hw * dinv_ref[...]
    hs_ref[N:, :] = jnp.zeros((NPAD - N, D), jnp.float32)


def _bn_relu(sp_ref, hw_ref, dinv_ref, g_ref, bt_ref):
    sagg = sp_ref[:N, :] + sp_ref[NPAD:NPAD + N, :]
    dinv = dinv_ref[...]
    z = dinv * sagg + (dinv * dinv) * hw_ref[...]
    mu = jnp.mean(z, axis=0, keepdims=True)
    zc = z - mu
    var = jnp.mean(zc * zc, axis=0, keepdims=True)
    return jnp.maximum(
        zc * lax.rsqrt(var + 1e-5) * g_ref[...] + bt_ref[...], 0.0)


def _tmid_body(sp_ref, hw_ref, dinv_ref, g_ref, bt_ref, wn_ref,
               hwn_ref, hsn_ref):
    h = _bn_relu(sp_ref, hw_ref, dinv_ref, g_ref, bt_ref)
    hw = jnp.dot(h, wn_ref[...], preferred_element_type=jnp.float32)
    hwn_ref[...] = hw
    hsn_ref[:N, :] = hw * dinv_ref[...]
    hsn_ref[N:, :] = jnp.zeros((NPAD - N, D), jnp.float32)


def _tfin_body(sp_ref, hw_ref, dinv_ref, g_ref, bt_ref, batch_ref, ef_ref,
               we1_ref, be1_ref, we2_ref, be2_ref, wf1_ref, bf1_ref,
               wf2_ref, bf2_ref, out_ref, xp_ref, comb_ref):
    h = _bn_relu(sp_ref, hw_ref, dinv_ref, g_ref, bt_ref)
    seg = lax.broadcasted_iota(jnp.int32, (G, N), 0)
    p = (batch_ref[...] == seg).astype(jnp.float32)
    sums = jnp.dot(p, h, preferred_element_type=jnp.float32)
    cnts = jnp.sum(p, axis=1, keepdims=True)
    xp = sums / jnp.maximum(cnts, 1.0)
    e = jnp.maximum(jnp.dot(ef_ref[...], we1_ref[...],
                            preferred_element_type=jnp.float32)
                    + be1_ref[...], 0.0)
    e = jnp.maximum(jnp.dot(e, we2_ref[...],
                            preferred_element_type=jnp.float32)
                    + be2_ref[...], 0.0)
    comb = jnp.maximum(
        jnp.dot(xp, wf1_ref[:D, :], preferred_element_type=jnp.float32)
        + jnp.dot(e, wf1_ref[D:, :], preferred_element_type=jnp.float32)
        + bf1_ref[...], 0.0)
    out_ref[...] = (jnp.dot(comb, wf2_ref[...],
                            preferred_element_type=jnp.float32)
                    + bf2_ref[...])
    xp_ref[...] = xp
    comb_ref[...] = comb


_t1_call = pl.pallas_call(
    _t1_body,
    out_shape=[jax.ShapeDtypeStruct((N, D), jnp.float32),
               jax.ShapeDtypeStruct((NPAD, D), jnp.float32)],
)

_tmid_call = pl.pallas_call(
    _tmid_body,
    out_shape=[jax.ShapeDtypeStruct((N, D), jnp.float32),
               jax.ShapeDtypeStruct((NPAD, D), jnp.float32)],
)

_tfin_call = pl.pallas_call(
    _tfin_body,
    out_shape=[
        jax.ShapeDtypeStruct((G, 1), jnp.float32),
        jax.ShapeDtypeStruct((G, D), jnp.float32),
        jax.ShapeDtypeStruct((G, D), jnp.float32),
    ],
)


def kernel(x, edge_index, batch, experimental_feat,
           W1, b1, g1, bt1, W2, b2, g2, bt2, W3, b3, g3, bt3,
           We1, be1, We2, be2, Wf1, bf1, Wf2, bf2):
    pad = EP - E
    packed = jnp.sort(edge_index[0] * 16384 + edge_index[1])
    src_s = packed >> 14
    dst_s = packed & 16383
    srcp = jnp.concatenate(
        [src_s, jnp.full((pad,), N - 1, edge_index.dtype)]).reshape(NW, CH, K)
    dstp = jnp.concatenate(
        [dst_s, jnp.full((pad,), N, edge_index.dtype)]).reshape(NW, CH, K)
    zeros1 = jnp.zeros((RPT,), jnp.float32)
    zeros2 = jnp.zeros((RPT, D), jnp.float32)

    gixp, rlp = _wprep_call(srcp.reshape(NW * CH, K))
    gixp = gixp.reshape(NW, CH, K)
    rlp = rlp.reshape(NW, CH, K)
    degp = _deg_call(dstp.reshape(NW, TPE // 128, 128), zeros1).reshape(NC, NPAD)
    dinv = lax.rsqrt(degp[0, :N] + degp[1, :N] + 1.0).reshape(N, 1)

    hw, hs = _t1_call(x, W1, dinv)
    for (g, bt, wn) in ((g1, bt1, W2), (g2, bt2, W3)):
        sp = _scat_call(hs, gixp, rlp, dstp, zeros2)
        hw, hs = _tmid_call(sp, hw, dinv, g.reshape(1, D), bt.reshape(1, D), wn)
    sp = _scat_call(hs, gixp, rlp, dstp, zeros2)
    out, xp, comb = _tfin_call(
        sp, hw, dinv, g3.reshape(1, D), bt3.reshape(1, D),
        batch.reshape(1, N), experimental_feat,
        We1, be1.reshape(1, -1), We2, be2.reshape(1, -1),
        Wf1, bf1.reshape(1, -1), Wf2, bf2.reshape(1, 1))
    return (out, xp, comb)
